# baseline (speedup 1.0000x reference)
import functools

import jax
import jax.numpy as jnp
from jax import lax
from jax.experimental import pallas as pl
from jax.experimental.pallas import tpu as pltpu
from jax.experimental.pallas import tpu_sc as plsc

_N = 50000
_E = 800000
_G = 128
_DIN = 128
_DH = 64
_DOUT = 32
_NS = 4
_W = _DH // _NS

_NSUB = 16
_DR_TILES = 10
_DR_ROWS = _N // _DR_TILES
_ZROWS = 625

_CH = 128
_DEG_NCH = _E // _CH // 32
_DEG_BASE = _DEG_NCH * _CH
_NCH = _E // _CH // _NSUB
_EDG_BASE = _NCH * _CH

_BN = 2000
_NBLK = _N // _BN

_mesh = plsc.VectorSubcoreMesh(
    core_axis_name="c", subcore_axis_name="s",
    num_cores=2, num_subcores=_NSUB)

_sc_params = pltpu.CompilerParams(use_tc_tiling_on_sc=False)


def _sc_degree(edge_index):
  out_type = (jax.ShapeDtypeStruct((_N, 16), jnp.float32),
              jax.ShapeDtypeStruct((_N, 16), jnp.float32))

  @functools.partial(
      pl.kernel, out_type=out_type, mesh=_mesh,
      compiler_params=_sc_params,
      scratch_types=[
          pltpu.VMEM((_ZROWS, 16), jnp.float32),
          pltpu.VMEM((2, 1, _CH), jnp.int32),
          pltpu.VMEM_SHARED((_N, 16), jnp.float32),
          pltpu.SemaphoreType.DMA((2,)),
      ])
  def deg_kernel(ei_hbm, out0, out1, buf_v, idx_v, acc_sh, si):
    c = lax.axis_index("c")
    s = lax.axis_index("s")
    base_r = s * _DR_ROWS

    @pl.loop(0, _ZROWS)
    def _(i):
      buf_v[i, :] = jnp.zeros((16,), jnp.float32)

    @pl.when(s < _DR_TILES)
    def _():
      @pl.loop(0, _DR_ROWS // _ZROWS)
      def _(j):
        pltpu.sync_copy(buf_v, acc_sh.at[pl.ds(base_r + j * _ZROWS, _ZROWS)])

    @pl.loop(0, _CH)
    def _(i):
      buf_v[i, :] = jnp.ones((16,), jnp.float32)

    plsc.subcore_barrier()

    w = c * _NSUB + s
    base_e = w * _DEG_BASE + jnp.minimum(w, 10) * _CH
    nch = jnp.where(w < 10, _DEG_NCH + 1, _DEG_NCH)

    def load_idx(k, slot):
      pltpu.async_copy(ei_hbm.at[1, pl.ds(base_e + k * _CH, _CH)],
                       idx_v.at[slot, 0], si.at[slot])

    def idx_wait(k, slot):
      pltpu.make_async_copy(ei_hbm.at[1, pl.ds(base_e + k * _CH, _CH)],
                            idx_v.at[slot, 0], si.at[slot]).wait()

    load_idx(0, 0)

    @pl.loop(0, nch)
    def _(i):
      p = lax.rem(i, 2)
      q = 1 - p

      @pl.when(i + 1 < nch)
      def _():
        load_idx(i + 1, q)

      idx_wait(i, p)
      pltpu.sync_copy(buf_v.at[pl.ds(0, _CH)], acc_sh.at[idx_v.at[p, 0]],
                      add=True)

    plsc.subcore_barrier()

    @pl.when(jnp.logical_and(c == 0, s < _DR_TILES))
    def _():
      pltpu.sync_copy(acc_sh.at[pl.ds(base_r, _DR_ROWS)],
                      out0.at[pl.ds(base_r, _DR_ROWS)])

    @pl.when(jnp.logical_and(c == 1, s < _DR_TILES))
    def _():
      pltpu.sync_copy(acc_sh.at[pl.ds(base_r, _DR_ROWS)],
                      out1.at[pl.ds(base_r, _DR_ROWS)])

  return deg_kernel(edge_index)


def _sc_edge_pass(edge_index, g0, g1, g2, g3):
  out_type = tuple(jax.ShapeDtypeStruct((_N, _W), jnp.float32)
                   for _ in range(4))

  @functools.partial(
      pl.kernel, out_type=out_type, mesh=_mesh,
      compiler_params=_sc_params,
      scratch_types=[
          pltpu.VMEM((8, 2, _CH), jnp.int32),
          pltpu.VMEM((8, _CH, _W), jnp.float32),
          pltpu.VMEM_SHARED((_N, _W), jnp.float32),
          pltpu.SemaphoreType.DMA((8,)),
          pltpu.SemaphoreType.DMA((8,)),
          pltpu.SemaphoreType.DMA((8,)),
      ])
  def edge_kernel(ei_hbm, g0_hbm, g1_hbm, g2_hbm, g3_hbm,
                  out0, out1, out2, out3,
                  eidx, rows_v, acc_sh, si, sg, ss):
    c = lax.axis_index("c")
    s = lax.axis_index("s")
    base_r = s * _DR_ROWS

    def init(g_hbm):
      @pl.when(s < _DR_TILES)
      def _():
        pltpu.sync_copy(g_hbm.at[pl.ds(base_r, _DR_ROWS)],
                        acc_sh.at[pl.ds(base_r, _DR_ROWS)])

    def run(g_hbm):
      base_e = s * _EDG_BASE + jnp.minimum(s, 10) * _CH
      nch = jnp.where(s < 10, _NCH + 1, _NCH)

      def load_idx(k, slot):
        pltpu.async_copy(ei_hbm.at[:, pl.ds(base_e + k * _CH, _CH)],
                         eidx.at[slot], si.at[slot])

      def idx_wait(k, slot):
        pltpu.make_async_copy(ei_hbm.at[:, pl.ds(base_e + k * _CH, _CH)],
                              eidx.at[slot], si.at[slot]).wait()

      def start_gather(slot):
        pltpu.async_copy(g_hbm.at[eidx.at[slot, 0]], rows_v.at[slot],
                         sg.at[slot])

      def gather_wait(slot):
        pltpu.make_async_copy(g_hbm.at[eidx.at[slot, 0]], rows_v.at[slot],
                              sg.at[slot]).wait()

      def start_scatter(slot):
        pltpu.async_copy(rows_v.at[slot], acc_sh.at[eidx.at[slot, 1]],
                         ss.at[slot], add=True)

      def scatter_wait(slot):
        pltpu.make_async_copy(rows_v.at[slot], acc_sh.at[eidx.at[slot, 1]],
                              ss.at[slot]).wait()

      for k in range(6):
        load_idx(k, k)
      for k in range(3):
        idx_wait(k, k)
        start_gather(k)

      @pl.loop(0, nch)
      def _(i):
        p = lax.rem(i, 8)

        @pl.when(i + 3 < nch)
        def _():
          q = lax.rem(i + 3, 8)
          idx_wait(i + 3, q)
          start_gather(q)

        gather_wait(p)
        start_scatter(p)

        @pl.when(i + 6 < nch)
        def _():
          r = lax.rem(i + 6, 8)

          @pl.when(i >= 2)
          def _():
            scatter_wait(r)

          load_idx(i + 6, r)

      @pl.loop(jnp.maximum(nch - 8, 0), nch)
      def _(k):
        scatter_wait(lax.rem(k, 8))

    def drain(out_hbm):
      @pl.when(s < _DR_TILES)
      def _():
        pltpu.sync_copy(acc_sh.at[pl.ds(base_r, _DR_ROWS)],
                        out_hbm.at[pl.ds(base_r, _DR_ROWS)])

    def do_round(g_hbm, out_hbm):
      init(g_hbm)
      plsc.subcore_barrier()
      run(g_hbm)
      plsc.subcore_barrier()
      drain(out_hbm)

    @pl.when(c == 0)
    def _():
      do_round(g0_hbm, out0)
      do_round(g2_hbm, out2)

    @pl.when(c == 1)
    def _():
      do_round(g1_hbm, out1)
      do_round(g3_hbm, out3)

  return edge_kernel(edge_index, g0, g1, g2, g3)


def _tc_matmul1(x, W1):
  def body(x_ref, w_ref, o_ref):
    o_ref[...] = jnp.dot(x_ref[...].astype(jnp.bfloat16),
                         w_ref[...].astype(jnp.bfloat16),
                         preferred_element_type=jnp.float32)

  return pl.pallas_call(
      body,
      grid=(_NBLK,),
      in_specs=[pl.BlockSpec((_BN, _DIN), lambda i: (i, 0)),
                pl.BlockSpec((_DIN, _DH), lambda i: (0, 0))],
      out_specs=pl.BlockSpec((_BN, _DH), lambda i: (i, 0)),
      out_shape=jax.ShapeDtypeStruct((_N, _DH), jnp.float32),
  )(x, W1)


def _tc_scale1(h1raw, deg0, deg1):
  def body(h_ref, d0_ref, d1_ref, g0_ref, g1_ref, g2_ref, g3_ref, dinv_ref):
    deg = d0_ref[:, 0:1] + d1_ref[:, 0:1] + 1.0
    dinv = 1.0 / jnp.sqrt(deg)
    g = h_ref[...] * dinv
    g0_ref[...] = g[:, 0 * _W:1 * _W]
    g1_ref[...] = g[:, 1 * _W:2 * _W]
    g2_ref[...] = g[:, 2 * _W:3 * _W]
    g3_ref[...] = g[:, 3 * _W:4 * _W]
    dinv_ref[...] = dinv

  slice_spec = pl.BlockSpec((_BN, _W), lambda i: (i, 0))
  slice_shape = jax.ShapeDtypeStruct((_N, _W), jnp.float32)
  return pl.pallas_call(
      body,
      grid=(_NBLK,),
      in_specs=[pl.BlockSpec((_BN, _DH), lambda i: (i, 0)),
                pl.BlockSpec((_BN, 16), lambda i: (i, 0)),
                pl.BlockSpec((_BN, 16), lambda i: (i, 0))],
      out_specs=[slice_spec] * 4 + [pl.BlockSpec((_BN, 1), lambda i: (i, 0))],
      out_shape=[slice_shape] * 4 + [jax.ShapeDtypeStruct((_N, 1),
                                                          jnp.float32)],
  )(h1raw, deg0, deg1)


def _tc_layer2(accs, dinv, W2, b1):
  def body(a0_ref, a1_ref, a2_ref, a3_ref, dinv_ref, w_ref, b_ref,
           g0_ref, g1_ref, g2_ref, g3_ref):
    dinv = dinv_ref[...]
    acc = jnp.concatenate(
        [a0_ref[...], a1_ref[...], a2_ref[...], a3_ref[...]], axis=1)
    h1 = jax.nn.relu(acc * dinv + b_ref[...])
    g2 = jnp.dot(h1.astype(jnp.bfloat16), w_ref[...].astype(jnp.bfloat16),
                 preferred_element_type=jnp.float32) * dinv
    g0_ref[...] = g2[:, 0 * _W:1 * _W]
    g1_ref[...] = g2[:, 1 * _W:2 * _W]
    g2_ref[...] = g2[:, 2 * _W:3 * _W]
    g3_ref[...] = g2[:, 3 * _W:4 * _W]

  slice_spec = pl.BlockSpec((_BN, _W), lambda i: (i, 0))
  slice_shape = jax.ShapeDtypeStruct((_N, _W), jnp.float32)
  return pl.pallas_call(
      body,
      grid=(_NBLK,),
      in_specs=[slice_spec] * 4 +
               [pl.BlockSpec((_BN, 1), lambda i: (i, 0)),
                pl.BlockSpec((_DH, _DH), lambda i: (0, 0)),
                pl.BlockSpec((1, _DH), lambda i: (0, 0))],
      out_specs=[slice_spec] * 4,
      out_shape=[slice_shape] * 4,
  )(*accs, dinv, W2, b1)


def _tc_finalize(accs, dinv, b2, batch2d, W_out, b_out):
  def body(a0_ref, a1_ref, a2_ref, a3_ref, dinv_ref, b2_ref, batch_ref,
           wo_ref, bo_ref, o_ref, sums_ref, counts_ref):
    i = pl.program_id(0)

    @pl.when(i == 0)
    def _():
      sums_ref[...] = jnp.zeros_like(sums_ref)
      counts_ref[...] = jnp.zeros_like(counts_ref)

    acc = jnp.concatenate(
        [a0_ref[...], a1_ref[...], a2_ref[...], a3_ref[...]], axis=1)
    h2 = jax.nn.relu(acc * dinv_ref[...] + b2_ref[...])
    gids = lax.broadcasted_iota(jnp.int32, (_BN, _G), 1)
    oh = jnp.where(batch_ref[...] == gids, 1.0, 0.0).astype(jnp.bfloat16)
    dn = (((0,), (0,)), ((), ()))
    sums_ref[...] += lax.dot_general(oh, h2.astype(jnp.bfloat16), dn,
                                     preferred_element_type=jnp.float32)
    counts_ref[...] += lax.dot_general(
        oh, jnp.ones((_BN, 1), jnp.bfloat16), dn,
        preferred_element_type=jnp.float32)

    @pl.when(i == _NBLK - 1)
    def _():
      pooled = sums_ref[...] / jnp.maximum(counts_ref[...], 1.0)
      o_ref[...] = jnp.dot(pooled, wo_ref[...],
                           preferred_element_type=jnp.float32) + bo_ref[...]

  slice_spec = pl.BlockSpec((_BN, _W), lambda i: (i, 0))
  return pl.pallas_call(
      body,
      grid=(_NBLK,),
      in_specs=[slice_spec] * 4 +
               [pl.BlockSpec((_BN, 1), lambda i: (i, 0)),
                pl.BlockSpec((1, _DH), lambda i: (0, 0)),
                pl.BlockSpec((_BN, 1), lambda i: (i, 0)),
                pl.BlockSpec((_DH, _DOUT), lambda i: (0, 0)),
                pl.BlockSpec((1, _DOUT), lambda i: (0, 0))],
      out_specs=pl.BlockSpec((_G, _DOUT), lambda i: (0, 0)),
      out_shape=jax.ShapeDtypeStruct((_G, _DOUT), jnp.float32),
      scratch_shapes=[pltpu.VMEM((_G, _DH), jnp.float32),
                      pltpu.VMEM((_G, 1), jnp.float32)],
  )(*accs, dinv, b2, batch2d, W_out, b_out)


@jax.jit
def kernel(x, edge_index, batch, W1, b1, W2, b2, W_out, b_out):
  batch2d = batch.reshape(_N, 1)
  b1r = b1.reshape(1, _DH)
  b2r = b2.reshape(1, _DH)
  b_outr = b_out.reshape(1, _DOUT)

  deg0, deg1 = _sc_degree(edge_index)
  h1raw = _tc_matmul1(x, W1)
  *g1s, dinv = _tc_scale1(h1raw, deg0, deg1)
  acc1 = _sc_edge_pass(edge_index, *g1s)
  g2s = _tc_layer2(acc1, dinv, W2, b1r)
  acc2 = _sc_edge_pass(edge_index, *g2s)
  return _tc_finalize(acc2, dinv, b2r, batch2d, W_out, b_outr)


# speedup vs baseline: 1.0997x; 1.0997x over previous
import functools

import jax
import jax.numpy as jnp
from jax import lax
from jax.experimental import pallas as pl
from jax.experimental.pallas import tpu as pltpu
from jax.experimental.pallas import tpu_sc as plsc

_N = 50000
_E = 800000
_G = 128
_DIN = 128
_DH = 64
_DOUT = 32
_NS = 4
_W = _DH // _NS

_NSUB = 16
_DR_TILES = 10
_DR_ROWS = _N // _DR_TILES
_ZROWS = 625

_CH = 128
_DEG_NCH = _E // _CH // 32
_DEG_BASE = _DEG_NCH * _CH
_NCH = _E // _CH // _NSUB
_EDG_BASE = _NCH * _CH

_BN = 2000
_NBLK = _N // _BN

_mesh = plsc.VectorSubcoreMesh(
    core_axis_name="c", subcore_axis_name="s",
    num_cores=2, num_subcores=_NSUB)

_sc_params = pltpu.CompilerParams(use_tc_tiling_on_sc=False)


def _sc_degree(edge_index):
  out_type = (jax.ShapeDtypeStruct((_N, 16), jnp.float32),
              jax.ShapeDtypeStruct((_N, 16), jnp.float32))

  @functools.partial(
      pl.kernel, out_type=out_type, mesh=_mesh,
      compiler_params=_sc_params,
      scratch_types=[
          pltpu.VMEM((_ZROWS, 16), jnp.float32),
          pltpu.VMEM((2, 1, _CH), jnp.int32),
          pltpu.VMEM_SHARED((_N, 16), jnp.float32),
          pltpu.SemaphoreType.DMA((2,)),
      ])
  def deg_kernel(ei_hbm, out0, out1, buf_v, idx_v, acc_sh, si):
    c = lax.axis_index("c")
    s = lax.axis_index("s")
    base_r = s * _DR_ROWS

    @pl.loop(0, _ZROWS)
    def _(i):
      buf_v[i, :] = jnp.zeros((16,), jnp.float32)

    @pl.when(s < _DR_TILES)
    def _():
      @pl.loop(0, _DR_ROWS // _ZROWS)
      def _(j):
        pltpu.sync_copy(buf_v, acc_sh.at[pl.ds(base_r + j * _ZROWS, _ZROWS)])

    @pl.loop(0, _CH)
    def _(i):
      buf_v[i, :] = jnp.ones((16,), jnp.float32)

    plsc.subcore_barrier()

    w = c * _NSUB + s
    base_e = w * _DEG_BASE + jnp.minimum(w, 10) * _CH
    nch = jnp.where(w < 10, _DEG_NCH + 1, _DEG_NCH)

    def load_idx(k, slot):
      pltpu.async_copy(ei_hbm.at[1, pl.ds(base_e + k * _CH, _CH)],
                       idx_v.at[slot, 0], si.at[slot])

    def idx_wait(k, slot):
      pltpu.make_async_copy(ei_hbm.at[1, pl.ds(base_e + k * _CH, _CH)],
                            idx_v.at[slot, 0], si.at[slot]).wait()

    load_idx(0, 0)

    @pl.loop(0, nch)
    def _(i):
      p = lax.rem(i, 2)
      q = 1 - p

      @pl.when(i + 1 < nch)
      def _():
        load_idx(i + 1, q)

      idx_wait(i, p)
      pltpu.sync_copy(buf_v.at[pl.ds(0, _CH)], acc_sh.at[idx_v.at[p, 0]],
                      add=True)

    plsc.subcore_barrier()

    @pl.when(jnp.logical_and(c == 0, s < _DR_TILES))
    def _():
      pltpu.sync_copy(acc_sh.at[pl.ds(base_r, _DR_ROWS)],
                      out0.at[pl.ds(base_r, _DR_ROWS)])

    @pl.when(jnp.logical_and(c == 1, s < _DR_TILES))
    def _():
      pltpu.sync_copy(acc_sh.at[pl.ds(base_r, _DR_ROWS)],
                      out1.at[pl.ds(base_r, _DR_ROWS)])

  return deg_kernel(edge_index)


def _sc_edge_pass(edge_index, g_a, g_b):
  out_type = jax.ShapeDtypeStruct((_N, 2 * _W), jnp.float32)

  @functools.partial(
      pl.kernel, out_type=out_type, mesh=_mesh,
      compiler_params=_sc_params,
      scratch_types=[
          pltpu.VMEM((8, 2, _CH), jnp.int32),
          pltpu.VMEM((8, _CH, _W), jnp.float32),
          pltpu.VMEM_SHARED((_N, _W), jnp.float32),
          pltpu.SemaphoreType.DMA((8,)),
          pltpu.SemaphoreType.DMA((8,)),
          pltpu.SemaphoreType.DMA((8,)),
      ])
  def edge_kernel(ei_hbm, ga_hbm, gb_hbm, out_hbm,
                  eidx, rows_v, acc_sh, si, sg, ss):
    c = lax.axis_index("c")
    s = lax.axis_index("s")
    base_r = s * _DR_ROWS

    def init(g_hbm):
      @pl.when(s < _DR_TILES)
      def _():
        pltpu.sync_copy(g_hbm.at[pl.ds(base_r, _DR_ROWS)],
                        acc_sh.at[pl.ds(base_r, _DR_ROWS)])

    def run(g_hbm):
      base_e = s * _EDG_BASE + jnp.minimum(s, 10) * _CH
      nch = jnp.where(s < 10, _NCH + 1, _NCH)

      def load_idx(k, slot):
        pltpu.async_copy(ei_hbm.at[:, pl.ds(base_e + k * _CH, _CH)],
                         eidx.at[slot], si.at[slot])

      def idx_wait(k, slot):
        pltpu.make_async_copy(ei_hbm.at[:, pl.ds(base_e + k * _CH, _CH)],
                              eidx.at[slot], si.at[slot]).wait()

      def start_gather(slot):
        pltpu.async_copy(g_hbm.at[eidx.at[slot, 0]], rows_v.at[slot],
                         sg.at[slot])

      def gather_wait(slot):
        pltpu.make_async_copy(g_hbm.at[eidx.at[slot, 0]], rows_v.at[slot],
                              sg.at[slot]).wait()

      def start_scatter(slot):
        pltpu.async_copy(rows_v.at[slot], acc_sh.at[eidx.at[slot, 1]],
                         ss.at[slot], add=True)

      def scatter_wait(slot):
        pltpu.make_async_copy(rows_v.at[slot], acc_sh.at[eidx.at[slot, 1]],
                              ss.at[slot]).wait()

      for k in range(6):
        load_idx(k, k)
      for k in range(3):
        idx_wait(k, k)
        start_gather(k)

      @pl.loop(0, nch)
      def _(i):
        p = lax.rem(i, 8)

        @pl.when(i + 3 < nch)
        def _():
          q = lax.rem(i + 3, 8)
          idx_wait(i + 3, q)
          start_gather(q)

        gather_wait(p)
        start_scatter(p)

        @pl.when(i + 6 < nch)
        def _():
          r = lax.rem(i + 6, 8)

          @pl.when(i >= 2)
          def _():
            scatter_wait(r)

          load_idx(i + 6, r)

      @pl.loop(jnp.maximum(nch - 8, 0), nch)
      def _(k):
        scatter_wait(lax.rem(k, 8))

    def drain(col0):
      @pl.when(s < _DR_TILES)
      def _():
        pltpu.sync_copy(acc_sh.at[pl.ds(base_r, _DR_ROWS)],
                        out_hbm.at[pl.ds(base_r, _DR_ROWS), pl.ds(col0, _W)])

    @pl.when(c == 0)
    def _():
      init(ga_hbm)

    @pl.when(c == 1)
    def _():
      init(gb_hbm)

    plsc.subcore_barrier()

    @pl.when(c == 0)
    def _():
      run(ga_hbm)

    @pl.when(c == 1)
    def _():
      run(gb_hbm)

    plsc.subcore_barrier()

    @pl.when(c == 0)
    def _():
      drain(0)

    @pl.when(c == 1)
    def _():
      drain(_W)

  return edge_kernel(edge_index, g_a, g_b)


def _tc_matmul1(x, W1):
  def body(x_ref, w_ref, o_ref):
    o_ref[...] = jnp.dot(x_ref[...].astype(jnp.bfloat16),
                         w_ref[...].astype(jnp.bfloat16),
                         preferred_element_type=jnp.float32)

  return pl.pallas_call(
      body,
      grid=(_NBLK,),
      in_specs=[pl.BlockSpec((_BN, _DIN), lambda i: (i, 0)),
                pl.BlockSpec((_DIN, _DH), lambda i: (0, 0))],
      out_specs=pl.BlockSpec((_BN, _DH), lambda i: (i, 0)),
      out_shape=jax.ShapeDtypeStruct((_N, _DH), jnp.float32),
  )(x, W1)


def _tc_scale1(h1raw, deg0, deg1):
  def body(h_ref, d0_ref, d1_ref, g0_ref, g1_ref, g2_ref, g3_ref, dinv_ref):
    deg = d0_ref[:, 0:1] + d1_ref[:, 0:1] + 1.0
    dinv = 1.0 / jnp.sqrt(deg)
    g = h_ref[...] * dinv
    g0_ref[...] = g[:, 0 * _W:1 * _W]
    g1_ref[...] = g[:, 1 * _W:2 * _W]
    g2_ref[...] = g[:, 2 * _W:3 * _W]
    g3_ref[...] = g[:, 3 * _W:4 * _W]
    dinv_ref[...] = dinv

  slice_spec = pl.BlockSpec((_BN, _W), lambda i: (i, 0))
  slice_shape = jax.ShapeDtypeStruct((_N, _W), jnp.float32)
  return pl.pallas_call(
      body,
      grid=(_NBLK,),
      in_specs=[pl.BlockSpec((_BN, _DH), lambda i: (i, 0)),
                pl.BlockSpec((_BN, 16), lambda i: (i, 0)),
                pl.BlockSpec((_BN, 16), lambda i: (i, 0))],
      out_specs=[slice_spec] * 4 + [pl.BlockSpec((_BN, 1), lambda i: (i, 0))],
      out_shape=[slice_shape] * 4 + [jax.ShapeDtypeStruct((_N, 1),
                                                          jnp.float32)],
  )(h1raw, deg0, deg1)


def _tc_layer2(accs, dinv, W2, b1):
  def body(a0_ref, a1_ref, dinv_ref, w_ref, b_ref,
           g0_ref, g1_ref, g2_ref, g3_ref):
    dinv = dinv_ref[...]
    acc = jnp.concatenate([a0_ref[...], a1_ref[...]], axis=1)
    h1 = jax.nn.relu(acc * dinv + b_ref[...])
    g2 = jnp.dot(h1.astype(jnp.bfloat16), w_ref[...].astype(jnp.bfloat16),
                 preferred_element_type=jnp.float32) * dinv
    g0_ref[...] = g2[:, 0 * _W:1 * _W]
    g1_ref[...] = g2[:, 1 * _W:2 * _W]
    g2_ref[...] = g2[:, 2 * _W:3 * _W]
    g3_ref[...] = g2[:, 3 * _W:4 * _W]

  slice_spec = pl.BlockSpec((_BN, _W), lambda i: (i, 0))
  half_spec = pl.BlockSpec((_BN, 2 * _W), lambda i: (i, 0))
  slice_shape = jax.ShapeDtypeStruct((_N, _W), jnp.float32)
  return pl.pallas_call(
      body,
      grid=(_NBLK,),
      in_specs=[half_spec] * 2 +
               [pl.BlockSpec((_BN, 1), lambda i: (i, 0)),
                pl.BlockSpec((_DH, _DH), lambda i: (0, 0)),
                pl.BlockSpec((1, _DH), lambda i: (0, 0))],
      out_specs=[slice_spec] * 4,
      out_shape=[slice_shape] * 4,
  )(*accs, dinv, W2, b1)


def _tc_finalize(accs, dinv, b2, batch2d, W_out, b_out):
  def body(a0_ref, a1_ref, dinv_ref, b2_ref, batch_ref,
           wo_ref, bo_ref, o_ref, sums_ref, counts_ref):
    i = pl.program_id(0)

    @pl.when(i == 0)
    def _():
      sums_ref[...] = jnp.zeros_like(sums_ref)
      counts_ref[...] = jnp.zeros_like(counts_ref)

    acc = jnp.concatenate([a0_ref[...], a1_ref[...]], axis=1)
    h2 = jax.nn.relu(acc * dinv_ref[...] + b2_ref[...])
    gids = lax.broadcasted_iota(jnp.int32, (_BN, _G), 1)
    oh = jnp.where(batch_ref[...] == gids, 1.0, 0.0).astype(jnp.bfloat16)
    dn = (((0,), (0,)), ((), ()))
    sums_ref[...] += lax.dot_general(oh, h2.astype(jnp.bfloat16), dn,
                                     preferred_element_type=jnp.float32)
    counts_ref[...] += lax.dot_general(
        oh, jnp.ones((_BN, 1), jnp.bfloat16), dn,
        preferred_element_type=jnp.float32)

    @pl.when(i == _NBLK - 1)
    def _():
      pooled = sums_ref[...] / jnp.maximum(counts_ref[...], 1.0)
      o_ref[...] = jnp.dot(pooled, wo_ref[...],
                           preferred_element_type=jnp.float32) + bo_ref[...]

  half_spec = pl.BlockSpec((_BN, 2 * _W), lambda i: (i, 0))
  return pl.pallas_call(
      body,
      grid=(_NBLK,),
      in_specs=[half_spec] * 2 +
               [pl.BlockSpec((_BN, 1), lambda i: (i, 0)),
                pl.BlockSpec((1, _DH), lambda i: (0, 0)),
                pl.BlockSpec((_BN, 1), lambda i: (i, 0)),
                pl.BlockSpec((_DH, _DOUT), lambda i: (0, 0)),
                pl.BlockSpec((1, _DOUT), lambda i: (0, 0))],
      out_specs=pl.BlockSpec((_G, _DOUT), lambda i: (0, 0)),
      out_shape=jax.ShapeDtypeStruct((_G, _DOUT), jnp.float32),
      scratch_shapes=[pltpu.VMEM((_G, _DH), jnp.float32),
                      pltpu.VMEM((_G, 1), jnp.float32)],
  )(*accs, dinv, b2, batch2d, W_out, b_out)


@jax.jit
def kernel(x, edge_index, batch, W1, b1, W2, b2, W_out, b_out):
  batch2d = batch.reshape(_N, 1)
  b1r = b1.reshape(1, _DH)
  b2r = b2.reshape(1, _DH)
  b_outr = b_out.reshape(1, _DOUT)

  deg0, deg1 = _sc_degree(edge_index)
  h1raw = _tc_matmul1(x, W1)
  *g1s, dinv = _tc_scale1(h1raw, deg0, deg1)
  acc1 = (_sc_edge_pass(edge_index, g1s[0], g1s[1]),
          _sc_edge_pass(edge_index, g1s[2], g1s[3]))
  g2s = _tc_layer2(acc1, dinv, W2, b1r)
  acc2 = (_sc_edge_pass(edge_index, g2s[0], g2s[1]),
          _sc_edge_pass(edge_index, g2s[2], g2s[3]))
  return _tc_finalize(acc2, dinv, b2r, batch2d, W_out, b_outr)


# speedup vs baseline: 1.1087x; 1.0082x over previous
import functools

import jax
import jax.numpy as jnp
from jax import lax
from jax.experimental import pallas as pl
from jax.experimental.pallas import tpu as pltpu
from jax.experimental.pallas import tpu_sc as plsc

_N = 50000
_E = 800000
_G = 128
_DIN = 128
_DH = 64
_DOUT = 32
_NS = 4
_W = _DH // _NS

_NSUB = 16
_DR_TILES = 10
_DR_ROWS = _N // _DR_TILES
_ZROWS = 625

_CH = 128
_DEG_NCH = _E // _CH // 32
_DEG_BASE = _DEG_NCH * _CH
_NCH = _E // _CH // _NSUB
_EDG_BASE = _NCH * _CH

_BN = 2000
_NBLK = _N // _BN

_mesh = plsc.VectorSubcoreMesh(
    core_axis_name="c", subcore_axis_name="s",
    num_cores=2, num_subcores=_NSUB)

_sc_params = pltpu.CompilerParams(use_tc_tiling_on_sc=False)


def _sc_degree(edge_index):
  out_type = jax.ShapeDtypeStruct((_N, 32), jnp.float32)

  @functools.partial(
      pl.kernel, out_type=out_type, mesh=_mesh,
      compiler_params=_sc_params,
      scratch_types=[
          pltpu.VMEM((_ZROWS, 16), jnp.float32),
          pltpu.VMEM((2, 1, _CH), jnp.int32),
          pltpu.VMEM_SHARED((_N, 16), jnp.float32),
          pltpu.SemaphoreType.DMA((2,)),
      ])
  def deg_kernel(ei_hbm, out_hbm, buf_v, idx_v, acc_sh, si):
    c = lax.axis_index("c")
    s = lax.axis_index("s")
    base_r = s * _DR_ROWS

    @pl.loop(0, _ZROWS)
    def _(i):
      buf_v[i, :] = jnp.zeros((16,), jnp.float32)

    @pl.when(s < _DR_TILES)
    def _():
      @pl.loop(0, _DR_ROWS // _ZROWS)
      def _(j):
        pltpu.sync_copy(buf_v, acc_sh.at[pl.ds(base_r + j * _ZROWS, _ZROWS)])

    @pl.loop(0, _CH)
    def _(i):
      buf_v[i, :] = jnp.ones((16,), jnp.float32)

    plsc.subcore_barrier()

    w = c * _NSUB + s
    base_e = w * _DEG_BASE + jnp.minimum(w, 10) * _CH
    nch = jnp.where(w < 10, _DEG_NCH + 1, _DEG_NCH)

    def load_idx(k, slot):
      pltpu.async_copy(ei_hbm.at[1, pl.ds(base_e + k * _CH, _CH)],
                       idx_v.at[slot, 0], si.at[slot])

    def idx_wait(k, slot):
      pltpu.make_async_copy(ei_hbm.at[1, pl.ds(base_e + k * _CH, _CH)],
                            idx_v.at[slot, 0], si.at[slot]).wait()

    load_idx(0, 0)

    @pl.loop(0, nch)
    def _(i):
      p = lax.rem(i, 2)
      q = 1 - p

      @pl.when(i + 1 < nch)
      def _():
        load_idx(i + 1, q)

      idx_wait(i, p)
      pltpu.sync_copy(buf_v.at[pl.ds(0, _CH)], acc_sh.at[idx_v.at[p, 0]],
                      add=True)

    plsc.subcore_barrier()

    @pl.when(jnp.logical_and(c == 0, s < _DR_TILES))
    def _():
      pltpu.sync_copy(acc_sh.at[pl.ds(base_r, _DR_ROWS)],
                      out_hbm.at[pl.ds(base_r, _DR_ROWS), pl.ds(0, 16)])

    @pl.when(jnp.logical_and(c == 1, s < _DR_TILES))
    def _():
      pltpu.sync_copy(acc_sh.at[pl.ds(base_r, _DR_ROWS)],
                      out_hbm.at[pl.ds(base_r, _DR_ROWS), pl.ds(16, 16)])

  return deg_kernel(edge_index)


def _sc_edge_pass(edge_index, g_a, g_b):
  out_type = jax.ShapeDtypeStruct((_N, 2 * _W), jnp.float32)

  @functools.partial(
      pl.kernel, out_type=out_type, mesh=_mesh,
      compiler_params=_sc_params,
      scratch_types=[
          pltpu.VMEM((8, 2, _CH), jnp.int32),
          pltpu.VMEM((8, _CH, _W), jnp.float32),
          pltpu.VMEM_SHARED((_N, _W), jnp.float32),
          pltpu.SemaphoreType.DMA((8,)),
          pltpu.SemaphoreType.DMA((8,)),
          pltpu.SemaphoreType.DMA((8,)),
      ])
  def edge_kernel(ei_hbm, ga_hbm, gb_hbm, out_hbm,
                  eidx, rows_v, acc_sh, si, sg, ss):
    c = lax.axis_index("c")
    s = lax.axis_index("s")
    base_r = s * _DR_ROWS

    def init(g_hbm):
      @pl.when(s < _DR_TILES)
      def _():
        pltpu.sync_copy(g_hbm.at[pl.ds(base_r, _DR_ROWS)],
                        acc_sh.at[pl.ds(base_r, _DR_ROWS)])

    def run(g_hbm):
      base_e = s * _EDG_BASE + jnp.minimum(s, 10) * _CH
      nch = jnp.where(s < 10, _NCH + 1, _NCH)

      def load_idx(k, slot):
        pltpu.async_copy(ei_hbm.at[:, pl.ds(base_e + k * _CH, _CH)],
                         eidx.at[slot], si.at[slot])

      def idx_wait(k, slot):
        pltpu.make_async_copy(ei_hbm.at[:, pl.ds(base_e + k * _CH, _CH)],
                              eidx.at[slot], si.at[slot]).wait()

      def start_gather(slot):
        pltpu.async_copy(g_hbm.at[eidx.at[slot, 0]], rows_v.at[slot],
                         sg.at[slot])

      def gather_wait(slot):
        pltpu.make_async_copy(g_hbm.at[eidx.at[slot, 0]], rows_v.at[slot],
                              sg.at[slot]).wait()

      def start_scatter(slot):
        pltpu.async_copy(rows_v.at[slot], acc_sh.at[eidx.at[slot, 1]],
                         ss.at[slot], add=True)

      def scatter_wait(slot):
        pltpu.make_async_copy(rows_v.at[slot], acc_sh.at[eidx.at[slot, 1]],
                              ss.at[slot]).wait()

      for k in range(6):
        load_idx(k, k)
      for k in range(3):
        idx_wait(k, k)
        start_gather(k)

      @pl.loop(0, nch)
      def _(i):
        p = lax.rem(i, 8)

        @pl.when(i + 3 < nch)
        def _():
          q = lax.rem(i + 3, 8)
          idx_wait(i + 3, q)
          start_gather(q)

        gather_wait(p)
        start_scatter(p)

        @pl.when(i + 6 < nch)
        def _():
          r = lax.rem(i + 6, 8)

          @pl.when(i >= 2)
          def _():
            scatter_wait(r)

          load_idx(i + 6, r)

      @pl.loop(jnp.maximum(nch - 8, 0), nch)
      def _(k):
        scatter_wait(lax.rem(k, 8))

    def drain(col0):
      @pl.when(s < _DR_TILES)
      def _():
        pltpu.sync_copy(acc_sh.at[pl.ds(base_r, _DR_ROWS)],
                        out_hbm.at[pl.ds(base_r, _DR_ROWS), pl.ds(col0, _W)])

    @pl.when(c == 0)
    def _():
      init(ga_hbm)

    @pl.when(c == 1)
    def _():
      init(gb_hbm)

    plsc.subcore_barrier()

    @pl.when(c == 0)
    def _():
      run(ga_hbm)

    @pl.when(c == 1)
    def _():
      run(gb_hbm)

    plsc.subcore_barrier()

    @pl.when(c == 0)
    def _():
      drain(0)

    @pl.when(c == 1)
    def _():
      drain(_W)

  return edge_kernel(edge_index, g_a, g_b)


def _tc_matmul1(x, W1):
  def body(x_ref, w_ref, o_ref):
    o_ref[...] = jnp.dot(x_ref[...].astype(jnp.bfloat16),
                         w_ref[...].astype(jnp.bfloat16),
                         preferred_element_type=jnp.float32)

  return pl.pallas_call(
      body,
      grid=(_NBLK,),
      in_specs=[pl.BlockSpec((_BN, _DIN), lambda i: (i, 0)),
                pl.BlockSpec((_DIN, _DH), lambda i: (0, 0))],
      out_specs=pl.BlockSpec((_BN, _DH), lambda i: (i, 0)),
      out_shape=jax.ShapeDtypeStruct((_N, _DH), jnp.float32),
  )(x, W1)


def _tc_scale1(h1raw, degs):
  def body(h_ref, d_ref, g0_ref, g1_ref, g2_ref, g3_ref, dinv_ref):
    deg = d_ref[:, 0:1] + d_ref[:, 16:17] + 1.0
    dinv = 1.0 / jnp.sqrt(deg)
    g = h_ref[...] * dinv
    g0_ref[...] = g[:, 0 * _W:1 * _W]
    g1_ref[...] = g[:, 1 * _W:2 * _W]
    g2_ref[...] = g[:, 2 * _W:3 * _W]
    g3_ref[...] = g[:, 3 * _W:4 * _W]
    dinv_ref[...] = dinv

  slice_spec = pl.BlockSpec((_BN, _W), lambda i: (i, 0))
  slice_shape = jax.ShapeDtypeStruct((_N, _W), jnp.float32)
  return pl.pallas_call(
      body,
      grid=(_NBLK,),
      in_specs=[pl.BlockSpec((_BN, _DH), lambda i: (i, 0)),
                pl.BlockSpec((_BN, 32), lambda i: (i, 0))],
      out_specs=[slice_spec] * 4 + [pl.BlockSpec((_BN, 1), lambda i: (i, 0))],
      out_shape=[slice_shape] * 4 + [jax.ShapeDtypeStruct((_N, 1),
                                                          jnp.float32)],
  )(h1raw, degs)


def _tc_layer2(accs, dinv, W2, b1):
  def body(a0_ref, a1_ref, dinv_ref, w_ref, b_ref,
           g0_ref, g1_ref, g2_ref, g3_ref):
    dinv = dinv_ref[...]
    acc = jnp.concatenate([a0_ref[...], a1_ref[...]], axis=1)
    h1 = jax.nn.relu(acc * dinv + b_ref[...])
    g2 = jnp.dot(h1.astype(jnp.bfloat16), w_ref[...].astype(jnp.bfloat16),
                 preferred_element_type=jnp.float32) * dinv
    g0_ref[...] = g2[:, 0 * _W:1 * _W]
    g1_ref[...] = g2[:, 1 * _W:2 * _W]
    g2_ref[...] = g2[:, 2 * _W:3 * _W]
    g3_ref[...] = g2[:, 3 * _W:4 * _W]

  slice_spec = pl.BlockSpec((_BN, _W), lambda i: (i, 0))
  half_spec = pl.BlockSpec((_BN, 2 * _W), lambda i: (i, 0))
  slice_shape = jax.ShapeDtypeStruct((_N, _W), jnp.float32)
  return pl.pallas_call(
      body,
      grid=(_NBLK,),
      in_specs=[half_spec] * 2 +
               [pl.BlockSpec((_BN, 1), lambda i: (i, 0)),
                pl.BlockSpec((_DH, _DH), lambda i: (0, 0)),
                pl.BlockSpec((1, _DH), lambda i: (0, 0))],
      out_specs=[slice_spec] * 4,
      out_shape=[slice_shape] * 4,
  )(*accs, dinv, W2, b1)


def _tc_finalize(accs, dinv, b2, batch2d, W_out, b_out):
  def body(a0_ref, a1_ref, dinv_ref, b2_ref, batch_ref,
           wo_ref, bo_ref, o_ref, sums_ref, counts_ref):
    i = pl.program_id(0)

    @pl.when(i == 0)
    def _():
      sums_ref[...] = jnp.zeros_like(sums_ref)
      counts_ref[...] = jnp.zeros_like(counts_ref)

    acc = jnp.concatenate([a0_ref[...], a1_ref[...]], axis=1)
    h2 = jax.nn.relu(acc * dinv_ref[...] + b2_ref[...])
    gids = lax.broadcasted_iota(jnp.int32, (_BN, _G), 1)
    oh = jnp.where(batch_ref[...] == gids, 1.0, 0.0).astype(jnp.bfloat16)
    dn = (((0,), (0,)), ((), ()))
    sums_ref[...] += lax.dot_general(oh, h2.astype(jnp.bfloat16), dn,
                                     preferred_element_type=jnp.float32)
    counts_ref[...] += lax.dot_general(
        oh, jnp.ones((_BN, 1), jnp.bfloat16), dn,
        preferred_element_type=jnp.float32)

    @pl.when(i == _NBLK - 1)
    def _():
      pooled = sums_ref[...] / jnp.maximum(counts_ref[...], 1.0)
      o_ref[...] = jnp.dot(pooled, wo_ref[...],
                           preferred_element_type=jnp.float32) + bo_ref[...]

  half_spec = pl.BlockSpec((_BN, 2 * _W), lambda i: (i, 0))
  return pl.pallas_call(
      body,
      grid=(_NBLK,),
      in_specs=[half_spec] * 2 +
               [pl.BlockSpec((_BN, 1), lambda i: (i, 0)),
                pl.BlockSpec((1, _DH), lambda i: (0, 0)),
                pl.BlockSpec((_BN, 1), lambda i: (i, 0)),
                pl.BlockSpec((_DH, _DOUT), lambda i: (0, 0)),
                pl.BlockSpec((1, _DOUT), lambda i: (0, 0))],
      out_specs=pl.BlockSpec((_G, _DOUT), lambda i: (0, 0)),
      out_shape=jax.ShapeDtypeStruct((_G, _DOUT), jnp.float32),
      scratch_shapes=[pltpu.VMEM((_G, _DH), jnp.float32),
                      pltpu.VMEM((_G, 1), jnp.float32)],
  )(*accs, dinv, b2, batch2d, W_out, b_out)


@jax.jit
def kernel(x, edge_index, batch, W1, b1, W2, b2, W_out, b_out):
  batch2d = batch.reshape(_N, 1)
  b1r = b1.reshape(1, _DH)
  b2r = b2.reshape(1, _DH)
  b_outr = b_out.reshape(1, _DOUT)

  degs = _sc_degree(edge_index)
  h1raw = _tc_matmul1(x, W1)
  *g1s, dinv = _tc_scale1(h1raw, degs)
  acc1 = (_sc_edge_pass(edge_index, g1s[0], g1s[1]),
          _sc_edge_pass(edge_index, g1s[2], g1s[3]))
  g2s = _tc_layer2(acc1, dinv, W2, b1r)
  acc2 = (_sc_edge_pass(edge_index, g2s[0], g2s[1]),
          _sc_edge_pass(edge_index, g2s[2], g2s[3]))
  return _tc_finalize(acc2, dinv, b2r, batch2d, W_out, b_outr)
